# baseline (speedup 1.0000x reference)
import functools

import jax
import jax.numpy as jnp
from jax import lax
from jax.experimental import pallas as pl
from jax.experimental.pallas import tpu as pltpu
from jax.experimental.pallas import tpu_sc as plsc

_NC = 2
_NS = 16
_NW = _NC * _NS
_L = 16

_CHUNK = 1024
_IPS = 128
_K = _CHUNK // _IPS


@functools.lru_cache(maxsize=None)
def _build(total, dim, fields):
    assert total % (_NW * _CHUNK) == 0
    per_w = total // _NW
    nchunk = per_w // _CHUNK
    mesh = plsc.VectorSubcoreMesh(core_axis_name="c", subcore_axis_name="s")

    @functools.partial(
        pl.kernel,
        out_type=jax.ShapeDtypeStruct((total, dim), jnp.float32),
        mesh=mesh,
        scratch_types=[
            pltpu.VMEM((_K, _IPS), jnp.int32),
            pltpu.VMEM((_CHUNK, dim), jnp.float32),
            pltpu.VMEM((_K, _IPS), jnp.int32),
            pltpu.SemaphoreType.DMA,
        ],
        compiler_params=pltpu.CompilerParams(use_tc_tiling_on_sc=False),
    )
    def gather_kernel(idx_hbm, off_hbm, table_hbm, out_hbm,
                      idx_v, rows_v, off_v, sem):
        wid = lax.axis_index("s") * _NC + lax.axis_index("c")
        base = wid * per_w

        def chunk_body(g, carry):
            cb = pl.multiple_of(base + g * _CHUNK, _CHUNK)
            rb = pl.multiple_of(cb // _IPS, _K)
            pltpu.sync_copy(idx_hbm.at[pl.ds(rb, _K)], idx_v)
            pltpu.sync_copy(off_hbm.at[pl.ds(rb, _K)], off_v)
            for j in range(_K):
                for i in range(_IPS // _L):
                    s = pl.ds(i * _L, _L)
                    idx_v[j, s] = idx_v[j, s] + off_v[j, s]
            descs = [
                pltpu.async_copy(table_hbm.at[idx_v.at[j]],
                                 rows_v.at[pl.ds(j * _IPS, _IPS)], sem)
                for j in range(_K)
            ]
            for d in descs:
                d.wait()
            pltpu.sync_copy(rows_v, out_hbm.at[pl.ds(cb, _CHUNK)])
            return carry

        lax.fori_loop(0, nchunk, chunk_body, 0)

    return gather_kernel


def kernel(input, num_classes, table):
    batch, fields = input.shape
    dim = table.shape[1]
    total = batch * fields
    offsets = jnp.concatenate([
        jnp.zeros((1,), dtype=num_classes.dtype),
        jnp.cumsum(num_classes)[:-1],
    ]).astype(jnp.int32)
    off2 = jnp.broadcast_to(offsets, (batch, fields)).reshape(
        total // _IPS, _IPS)
    idx2 = input.reshape(total // _IPS, _IPS)
    out = _build(total, dim, fields)(idx2, off2, table)
    return out.reshape(batch, fields * dim)


# speedup vs baseline: 1.0117x; 1.0117x over previous
import functools

import jax
import jax.numpy as jnp
from jax import lax
from jax.experimental import pallas as pl
from jax.experimental.pallas import tpu as pltpu
from jax.experimental.pallas import tpu_sc as plsc

_NC = 2
_NS = 16
_NW = _NC * _NS
_L = 16

_CHUNK = 1664
_IPS = 128
_K = _CHUNK // _IPS


@functools.lru_cache(maxsize=None)
def _build(total, dim):
    assert total % (_NW * _CHUNK) == 0
    per_w = total // _NW
    nchunk = per_w // _CHUNK
    assert nchunk % 2 == 0
    mesh = plsc.VectorSubcoreMesh(core_axis_name="c", subcore_axis_name="s")

    @functools.partial(
        pl.kernel,
        out_type=jax.ShapeDtypeStruct((total, dim), jnp.float32),
        mesh=mesh,
        scratch_types=[
            pltpu.VMEM((2, _K, _IPS), jnp.int32),
            pltpu.VMEM((2, _K, _IPS), jnp.int32),
            pltpu.VMEM((2, _CHUNK, dim), jnp.float32),
            pltpu.SemaphoreType.DMA,
            pltpu.SemaphoreType.DMA,
            pltpu.SemaphoreType.DMA,
            pltpu.SemaphoreType.DMA,
            pltpu.SemaphoreType.DMA,
            pltpu.SemaphoreType.DMA,
        ],
        compiler_params=pltpu.CompilerParams(use_tc_tiling_on_sc=False),
    )
    def gather_kernel(idx_hbm, off_hbm, table_hbm, out_hbm,
                      idx_v, off_v, rows_v,
                      sem_in0, sem_in1, sem_g0, sem_g1, sem_wb0, sem_wb1):
        sem_in = (sem_in0, sem_in1)
        sem_g = (sem_g0, sem_g1)
        sem_wb = (sem_wb0, sem_wb1)
        wid = lax.axis_index("s") * _NC + lax.axis_index("c")
        cbase = wid * nchunk

        def issue_in(gid, b):
            pltpu.async_copy(idx_hbm.at[gid], idx_v.at[b], sem_in[b])
            pltpu.async_copy(off_hbm.at[gid], off_v.at[b], sem_in[b])

        def wait_in(b):
            pltpu.make_async_copy(idx_hbm.at[0], idx_v.at[b], sem_in[b]).wait()
            pltpu.make_async_copy(off_hbm.at[0], off_v.at[b], sem_in[b]).wait()

        def adds(b):
            for j in range(_K):
                for i in range(_IPS // _L):
                    s = pl.ds(i * _L, _L)
                    idx_v[b, j, s] = idx_v[b, j, s] + off_v[b, j, s]

        def fire(b):
            for j in range(_K):
                pltpu.async_copy(table_hbm.at[idx_v.at[b, j]],
                                 rows_v.at[b, pl.ds(j * _IPS, _IPS)],
                                 sem_g[b])

        def wait_gathers(b):
            for j in range(_K):
                pltpu.make_async_copy(
                    table_hbm.at[idx_v.at[b, j]],
                    rows_v.at[b, pl.ds(j * _IPS, _IPS)], sem_g[b]).wait()

        def issue_wb(gid, b):
            cb = pl.multiple_of(gid * _CHUNK, 8)
            pltpu.async_copy(rows_v.at[b], out_hbm.at[pl.ds(cb, _CHUNK)],
                             sem_wb[b])

        def wait_wb(b):
            pltpu.make_async_copy(rows_v.at[b], out_hbm.at[pl.ds(0, _CHUNK)],
                                  sem_wb[b]).wait()

        issue_in(cbase, 0)
        issue_in(cbase + 1, 1)
        wait_in(0)
        adds(0)

        def body(t, carry):
            for p, b in ((0, 0), (1, 1)):
                gid = cbase + 2 * t + p
                @pl.when(t >= 1)
                def _():
                    wait_wb(b)
                fire(b)

                def prep():
                    wait_in(1 - b)
                    adds(1 - b)
                if p == 0:
                    prep()
                else:
                    pl.when(t < (nchunk // 2) - 1)(prep)
                wait_gathers(b)
                issue_wb(gid, b)
                @pl.when(t < (nchunk // 2) - 1)
                def _():
                    issue_in(gid + 2, b)
            return carry

        lax.fori_loop(0, nchunk // 2, body, 0)
        wait_wb(0)
        wait_wb(1)

    return gather_kernel


def kernel(input, num_classes, table):
    batch, fields = input.shape
    dim = table.shape[1]
    total = batch * fields
    offsets = jnp.concatenate([
        jnp.zeros((1,), dtype=num_classes.dtype),
        jnp.cumsum(num_classes)[:-1],
    ]).astype(jnp.int32)
    nblk = total // _CHUNK
    idx3 = input.reshape(nblk, _K, _IPS)
    off3 = jnp.broadcast_to(offsets, (batch, fields)).reshape(
        nblk, _K, _IPS)
    out = _build(total, dim)(idx3, off3, table)
    return out.reshape(batch, fields * dim)
